# baseline (speedup 1.0000x reference)
import jax
import jax.numpy as jnp
from jax import lax
from jax.experimental import pallas as pl
from jax.experimental.pallas import tpu as pltpu

_NE = 1024
_D = 64
_N_BLK = 8192


def _vq_body(z_ref, wm2_ref, w2_ref, wt_ref, zq_ref, idx_ref, sse_ref):
    zb = z_ref[0]
    wm2 = wm2_ref[...]
    w2 = w2_ref[...]
    wt = wt_ref[...]
    mT = lax.dot_general(wm2, zb, (((1,), (0,)), ((), ())),
                         preferred_element_type=jnp.float32)
    z2 = jnp.sum(zb * zb, axis=0)[None, :]
    d = (z2 + mT) + w2
    minv = jnp.min(d, axis=0, keepdims=True)
    iota = lax.broadcasted_iota(jnp.int32, (_NE, _N_BLK), 0)
    t = jnp.where(d == minv, iota, _NE)
    idx = jnp.min(t, axis=0)
    oh = jnp.where(t == idx[None, :], 1.0, 0.0)
    zq = lax.dot_general(wt, oh, (((1,), (0,)), ((), ())),
                         preferred_element_type=jnp.float32)
    zq_ref[0] = zb + (zq - zb)
    idx_ref[...] = idx.reshape(1, 1, 1, _N_BLK)
    diff = zq - zb
    p = jnp.sum(diff * diff)
    first = (pl.program_id(0) == 0) & (pl.program_id(1) == 0)

    @pl.when(first)
    def _():
        sse_ref[0, 0] = 0.0

    sse_ref[0, 0] = sse_ref[0, 0] + p


def kernel(z, W):
    B, C, T, H, Wd = z.shape
    S = T * H * Wd
    z3 = z.reshape(B, C, S)
    WT = W.T
    Wm2 = W * (-2.0)
    w2 = jnp.sum(W ** 2, axis=1)[:, None]
    nb = S // _N_BLK
    zq3, idx4, sse = pl.pallas_call(
        _vq_body,
        grid=(B, nb),
        in_specs=[
            pl.BlockSpec((1, C, _N_BLK), lambda b, n: (b, 0, n)),
            pl.BlockSpec((_NE, _D), lambda b, n: (0, 0)),
            pl.BlockSpec((_NE, 1), lambda b, n: (0, 0)),
            pl.BlockSpec((_D, _NE), lambda b, n: (0, 0)),
        ],
        out_specs=[
            pl.BlockSpec((1, C, _N_BLK), lambda b, n: (b, 0, n)),
            pl.BlockSpec((1, 1, 1, _N_BLK), lambda b, n: (b, n, 0, 0)),
            pl.BlockSpec(memory_space=pltpu.SMEM),
        ],
        out_shape=[
            jax.ShapeDtypeStruct((B, C, S), jnp.float32),
            jax.ShapeDtypeStruct((B, nb, 1, _N_BLK), jnp.int32),
            jax.ShapeDtypeStruct((1, 1), jnp.float32),
        ],
    )(z3, Wm2, w2, WT)
    zq_st = zq3.reshape(B, C, T, H, Wd)
    indices = idx4.reshape(B, T, H, Wd)
    mean = sse[0, 0] / (B * C * S)
    vq_loss = mean + 0.25 * mean
    return zq_st, vq_loss, indices


# speedup vs baseline: 1.2472x; 1.2472x over previous
import jax
import jax.numpy as jnp
from jax import lax
from jax.experimental import pallas as pl
from jax.experimental.pallas import tpu as pltpu

_NE = 1024
_D = 64
_N_BLK = 4096


def _vq_body(z_ref, wm2_ref, w2_ref, wt_ref, zq_ref, idx_ref, sse_ref):
    zb = z_ref[0]
    wm2 = wm2_ref[...]
    w2 = w2_ref[...]
    wt = wt_ref[...]
    mT = lax.dot_general(wm2, zb, (((1,), (0,)), ((), ())),
                         preferred_element_type=jnp.float32)
    z2 = jnp.sum(zb * zb, axis=0)[None, :]
    d = (z2 + mT) + w2
    idx = jnp.argmin(d, axis=0)
    iota = lax.broadcasted_iota(jnp.int32, (_NE, _N_BLK), 0)
    oh = jnp.where(iota == idx[None, :], 1.0, 0.0)
    zq = lax.dot_general(wt, oh, (((1,), (0,)), ((), ())),
                         preferred_element_type=jnp.float32)
    zq_ref[0] = zb + (zq - zb)
    idx_ref[...] = idx.reshape(1, 1, 1, _N_BLK)
    diff = zq - zb
    p = jnp.sum(diff * diff)
    first = (pl.program_id(0) == 0) & (pl.program_id(1) == 0)

    @pl.when(first)
    def _():
        sse_ref[0, 0] = 0.0

    sse_ref[0, 0] = sse_ref[0, 0] + p


def kernel(z, W):
    B, C, T, H, Wd = z.shape
    S = T * H * Wd
    z3 = z.reshape(B, C, S)
    WT = W.T
    Wm2 = W * (-2.0)
    w2 = jnp.sum(W ** 2, axis=1)[:, None]
    nb = S // _N_BLK
    zq3, idx4, sse = pl.pallas_call(
        _vq_body,
        grid=(B, nb),
        in_specs=[
            pl.BlockSpec((1, C, _N_BLK), lambda b, n: (b, 0, n)),
            pl.BlockSpec((_NE, _D), lambda b, n: (0, 0)),
            pl.BlockSpec((_NE, 1), lambda b, n: (0, 0)),
            pl.BlockSpec((_D, _NE), lambda b, n: (0, 0)),
        ],
        out_specs=[
            pl.BlockSpec((1, C, _N_BLK), lambda b, n: (b, 0, n)),
            pl.BlockSpec((1, 1, 1, _N_BLK), lambda b, n: (b, n, 0, 0)),
            pl.BlockSpec(memory_space=pltpu.SMEM),
        ],
        out_shape=[
            jax.ShapeDtypeStruct((B, C, S), jnp.float32),
            jax.ShapeDtypeStruct((B, nb, 1, _N_BLK), jnp.int32),
            jax.ShapeDtypeStruct((1, 1), jnp.float32),
        ],
    )(z3, Wm2, w2, WT)
    zq_st = zq3.reshape(B, C, T, H, Wd)
    indices = idx4.reshape(B, T, H, Wd)
    mean = sse[0, 0] / (B * C * S)
    vq_loss = mean + 0.25 * mean
    return zq_st, vq_loss, indices
